# baseline (speedup 1.0000x reference)
import functools

import jax
import jax.numpy as jnp
from jax import lax
from jax.experimental import pallas as pl
from jax.experimental.pallas import tpu as pltpu
from jax.experimental.pallas import tpu_sc as plsc

N = 10000
E = 320000
D = 128

NPAD = 10240
PADROW = 10200
NC = 2
NS = 16
NW = NC * NS
CH = 128
KCH = 80
EPW = CH * KCH
EPAD = EPW * NW
EROWS = EPAD // CH
FAST_CORE = 0
KFAST = 152
KSLOW = 8
QCH = 8
RPW = NPAD // NS

BM = 256
NBLK = NPAD // BM
EPS = 1e-5

_mesh = plsc.VectorSubcoreMesh(core_axis_name="c", subcore_axis_name="s")


def _make_deg(width):
    @functools.partial(
        pl.kernel,
        out_type=jax.ShapeDtypeStruct((NC, NPAD, width), jnp.float32),
        mesh=_mesh,
        scratch_types=[
            pltpu.VMEM((KCH, CH), jnp.int32),
            pltpu.VMEM((CH, width), jnp.float32),
            pltpu.VMEM_SHARED((NPAD, width), jnp.float32),
        ],
    )
    def _deg(dst2_hbm, ones_hbm, zeros_hbm, out_hbm, dstv, onesv, deg_sh):
        c = lax.axis_index("c")
        s = lax.axis_index("s")
        wid = s * NC + c
        pltpu.sync_copy(zeros_hbm, deg_sh.at[pl.ds(s * RPW, RPW)])
        pltpu.sync_copy(ones_hbm, onesv)
        pltpu.sync_copy(dst2_hbm.at[pl.ds(wid * KCH, KCH)], dstv)
        plsc.subcore_barrier()

        def body(j, carry):
            pltpu.sync_copy(onesv, deg_sh.at[dstv.at[j]], add=True)
            return carry

        lax.fori_loop(0, KCH, body, 0)
        plsc.subcore_barrier()
        pltpu.sync_copy(deg_sh.at[pl.ds(s * RPW, RPW)],
                        out_hbm.at[c].at[pl.ds(s * RPW, RPW)])

    return _deg


DEGW = 128
_sc_degree = _make_deg(DEGW)


@functools.partial(
    pl.kernel,
    out_type=jax.ShapeDtypeStruct((NC, NPAD, D), jnp.float32),
    mesh=_mesh,
    scratch_types=[
        pltpu.VMEM((QCH, CH), jnp.int32),
        pltpu.VMEM((QCH, CH), jnp.int32),
        pltpu.VMEM((CH, D), jnp.float32),
        pltpu.VMEM((CH, D), jnp.float32),
        pltpu.SemaphoreType.DMA,
        pltpu.VMEM_SHARED((NPAD, D), jnp.float32),
    ],
)
def _sc_spmm(hs_hbm, src2_hbm, dst2_hbm, zeros_hbm, out_hbm,
             srcv, dstv, rows0, rows1, sem, y_sh):
    c = lax.axis_index("c")
    s = lax.axis_index("s")
    pltpu.sync_copy(zeros_hbm, y_sh.at[pl.ds(s * RPW, RPW)])
    plsc.subcore_barrier()

    on_fast = c == FAST_CORE
    nq = jnp.where(on_fast, KFAST // QCH, KSLOW // QCH)
    off = jnp.where(on_fast, s * KFAST, NS * KFAST + s * KSLOW)

    def gath(j, buf):
        pltpu.async_copy(hs_hbm.at[srcv.at[j]], buf, sem)

    def gwait(buf):
        pltpu.make_async_copy(hs_hbm.at[srcv.at[0]], buf, sem).wait()

    def qbody(q, carry):
        base = off + q * QCH
        pltpu.sync_copy(src2_hbm.at[pl.ds(base, QCH)], srcv)
        pltpu.sync_copy(dst2_hbm.at[pl.ds(base, QCH)], dstv)

        gath(0, rows0)

        def body(jj, carry2):
            j0 = 2 * jj
            gwait(rows0)
            gath(j0 + 1, rows1)
            pltpu.sync_copy(rows0, y_sh.at[dstv.at[j0]], add=True)
            gwait(rows1)

            @pl.when(jj < QCH // 2 - 1)
            def _():
                gath(j0 + 2, rows0)

            pltpu.sync_copy(rows1, y_sh.at[dstv.at[j0 + 1]], add=True)
            return carry2

        lax.fori_loop(0, QCH // 2, body, 0)
        return carry

    lax.fori_loop(0, nq, qbody, 0)
    plsc.subcore_barrier()
    pltpu.sync_copy(y_sh.at[pl.ds(s * RPW, RPW)],
                    out_hbm.at[c].at[pl.ds(s * RPW, RPW)])


def _dis_block(deg_ref, i):
    p = deg_ref[...]
    deg = p[0, :, 0:1] + p[1, :, 0:1] + 1.0
    dis = lax.rsqrt(deg)
    rows = i * BM + lax.broadcasted_iota(jnp.int32, (BM, 1), 0)
    return jnp.where(rows < N, dis, 0.0)


def _k1_body(deg_ref, x_ref, w_ref, o_ref):
    i = pl.program_id(0)
    dis = _dis_block(deg_ref, i)
    o_ref[...] = dis * jnp.dot(x_ref[...], w_ref[...],
                               preferred_element_type=jnp.float32)


def _k2_body(deg_ref, y_ref, hs_ref, b_ref, o_ref, st_ref, acc_ref):
    i = pl.program_id(0)
    dis = _dis_block(deg_ref, i)
    o = dis * (y_ref[0] + y_ref[1] + hs_ref[...]) + b_ref[...]
    o_ref[...] = o

    @pl.when(i == 0)
    def _():
        acc_ref[...] = jnp.zeros_like(acc_ref)

    rows = i * BM + lax.broadcasted_iota(jnp.int32, (BM, 1), 0)
    om = jnp.where(rows < N, o, 0.0)
    acc_ref[0:1, :] += jnp.sum(om, axis=0, keepdims=True)
    acc_ref[1:2, :] += jnp.sum(om * om, axis=0, keepdims=True)

    @pl.when(i == NBLK - 1)
    def _():
        m = acc_ref[0:1, :] / N
        v = acc_ref[1:2, :] / N - m * m
        st_ref[...] = jnp.concatenate([m, lax.rsqrt(v + EPS)], axis=0)


def _bn_relu(o_ref, st_ref, g_ref, be_ref):
    st = st_ref[...]
    return jax.nn.relu((o_ref[...] - st[0:1, :]) * st[1:2, :]
                       * g_ref[...] + be_ref[...])


def _k3_body(deg_ref, o_ref, st_ref, g_ref, be_ref, w_ref, out_ref):
    i = pl.program_id(0)
    dis = _dis_block(deg_ref, i)
    h = _bn_relu(o_ref, st_ref, g_ref, be_ref)
    out_ref[...] = dis * jnp.dot(h, w_ref[...],
                                 preferred_element_type=jnp.float32)


def _kf_body(o_ref, st_ref, g_ref, be_ref, w1_ref, b1_ref, w2_ref, b2_ref,
             out_ref):
    h = _bn_relu(o_ref, st_ref, g_ref, be_ref)
    h = jax.nn.relu(jnp.dot(h, w1_ref[...],
                            preferred_element_type=jnp.float32) + b1_ref[...])
    o = jnp.dot(h, w2_ref[...],
                preferred_element_type=jnp.float32) + b2_ref[...]
    out_ref[...] = jax.nn.sigmoid(o)


def _full(shape):
    return pl.BlockSpec(shape, lambda i: tuple(0 for _ in shape))


_deg_spec = pl.BlockSpec((NC, BM, DEGW), lambda i: (0, i, 0))
_row_spec = pl.BlockSpec((BM, D), lambda i: (i, 0))


def _k1(degp, xpad, W):
    return pl.pallas_call(
        _k1_body,
        grid=(NBLK,),
        in_specs=[_deg_spec, _row_spec, _full((D, D))],
        out_specs=_row_spec,
        out_shape=jax.ShapeDtypeStruct((NPAD, D), jnp.float32),
    )(degp, xpad, W)


def _k2(degp, yp, hs, b):
    return pl.pallas_call(
        _k2_body,
        grid=(NBLK,),
        in_specs=[_deg_spec, pl.BlockSpec((NC, BM, D), lambda i: (0, i, 0)),
                  _row_spec, _full((1, D))],
        out_specs=[_row_spec, _full((2, D))],
        out_shape=[jax.ShapeDtypeStruct((NPAD, D), jnp.float32),
                   jax.ShapeDtypeStruct((2, D), jnp.float32)],
        scratch_shapes=[pltpu.VMEM((2, D), jnp.float32)],
    )(degp, yp, hs, b)


def _k3(degp, o, st, g, be, Wn):
    return pl.pallas_call(
        _k3_body,
        grid=(NBLK,),
        in_specs=[_deg_spec, _row_spec, _full((2, D)), _full((1, D)),
                  _full((1, D)), _full((D, D))],
        out_specs=_row_spec,
        out_shape=jax.ShapeDtypeStruct((NPAD, D), jnp.float32),
    )(degp, o, st, g, be, Wn)


def _kf(o, st, g, be, fc1_W, fc1_b, fc2_W, fc2_b):
    return pl.pallas_call(
        _kf_body,
        grid=(NBLK,),
        in_specs=[_row_spec, _full((2, D)), _full((1, D)), _full((1, D)),
                  _full((D, D // 2)), _full((1, D // 2)),
                  _full((D // 2, 1)), _full((1, 1))],
        out_specs=pl.BlockSpec((BM, 1), lambda i: (i, 0)),
        out_shape=jax.ShapeDtypeStruct((NPAD, 1), jnp.float32),
    )(o, st, g, be, fc1_W, fc1_b, fc2_W, fc2_b)


def kernel(x, edge_index, W1, b1, g1, be1, W2, b2, g2, be2, W3, b3, g3, be3,
           fc1_W, fc1_b, fc2_W, fc2_b):
    xpad = jnp.pad(x, ((0, NPAD - N), (0, 0)))
    pad = jnp.full((EPAD - E,), PADROW, jnp.int32)
    src2 = jnp.concatenate([edge_index[0], pad]).reshape(EROWS, CH)
    dst2 = jnp.concatenate([edge_index[1], pad]).reshape(EROWS, CH)
    zrowsD = jnp.zeros((RPW, D), jnp.float32)
    zrowsW = jnp.zeros((RPW, DEGW), jnp.float32)
    onesW = jnp.ones((CH, DEGW), jnp.float32)

    degp = _sc_degree(dst2, onesW, zrowsW)

    hs = _k1(degp, xpad, W1)
    params = [(b1, g1, be1), (b2, g2, be2), (b3, g3, be3)]
    nexts = [W2, W3, None]
    o = st = None
    for (b, g, be), Wn in zip(params, nexts):
        yp = _sc_spmm(hs, src2, dst2, zrowsD)
        o, st = _k2(degp, yp, hs, b.reshape(1, D))
        if Wn is not None:
            hs = _k3(degp, o, st, g.reshape(1, D), be.reshape(1, D), Wn)
    g3r, be3r = g3.reshape(1, D), be3.reshape(1, D)
    out = _kf(o, st, g3r, be3r, fc1_W, fc1_b.reshape(1, D // 2),
              fc2_W, fc2_b.reshape(1, 1))
    return out[:N]


# speedup vs baseline: 1.0273x; 1.0273x over previous
import functools

import jax
import jax.numpy as jnp
from jax import lax
from jax.experimental import pallas as pl
from jax.experimental.pallas import tpu as pltpu
from jax.experimental.pallas import tpu_sc as plsc

N = 10000
E = 320000
D = 128

NPAD = 10240
PADROW = 10200
NC = 2
NS = 16
NW = NC * NS
CH = 128
KCH = 80
EPW = CH * KCH
EPAD = EPW * NW
EROWS = EPAD // CH
FAST_CORE = 0
KFAST = 144
KSLOW = 16
QCH = 16
RPW = NPAD // NS

BM = 256
NBLK = NPAD // BM
EPS = 1e-5

_mesh = plsc.VectorSubcoreMesh(core_axis_name="c", subcore_axis_name="s")


def _make_deg(width):
    @functools.partial(
        pl.kernel,
        out_type=jax.ShapeDtypeStruct((NC, NPAD, width), jnp.float32),
        mesh=_mesh,
        scratch_types=[
            pltpu.VMEM((KCH, CH), jnp.int32),
            pltpu.VMEM((CH, width), jnp.float32),
            pltpu.VMEM_SHARED((NPAD, width), jnp.float32),
        ],
    )
    def _deg(dst2_hbm, ones_hbm, zeros_hbm, out_hbm, dstv, onesv, deg_sh):
        c = lax.axis_index("c")
        s = lax.axis_index("s")
        wid = s * NC + c
        pltpu.sync_copy(zeros_hbm, deg_sh.at[pl.ds(s * RPW, RPW)])
        pltpu.sync_copy(ones_hbm, onesv)
        pltpu.sync_copy(dst2_hbm.at[pl.ds(wid * KCH, KCH)], dstv)
        plsc.subcore_barrier()

        def body(j, carry):
            pltpu.sync_copy(onesv, deg_sh.at[dstv.at[j]], add=True)
            return carry

        lax.fori_loop(0, KCH, body, 0)
        plsc.subcore_barrier()
        pltpu.sync_copy(deg_sh.at[pl.ds(s * RPW, RPW)],
                        out_hbm.at[c].at[pl.ds(s * RPW, RPW)])

    return _deg


DEGW = 128
_sc_degree = _make_deg(DEGW)


@functools.partial(
    pl.kernel,
    out_type=jax.ShapeDtypeStruct((NC, NPAD, D), jnp.float32),
    mesh=_mesh,
    scratch_types=[
        pltpu.VMEM((QCH, CH), jnp.int32),
        pltpu.VMEM((QCH, CH), jnp.int32),
        pltpu.VMEM((CH, D), jnp.float32),
        pltpu.VMEM((CH, D), jnp.float32),
        pltpu.SemaphoreType.DMA,
        pltpu.VMEM_SHARED((NPAD, D), jnp.float32),
    ],
)
def _sc_spmm(hs_hbm, src2_hbm, dst2_hbm, zeros_hbm, out_hbm,
             srcv, dstv, rows0, rows1, sem, y_sh):
    c = lax.axis_index("c")
    s = lax.axis_index("s")
    pltpu.sync_copy(zeros_hbm, y_sh.at[pl.ds(s * RPW, RPW)])
    plsc.subcore_barrier()

    on_fast = c == FAST_CORE
    nq = jnp.where(on_fast, KFAST // QCH, KSLOW // QCH)
    off = jnp.where(on_fast, s * KFAST, NS * KFAST + s * KSLOW)

    def gath(j, buf):
        pltpu.async_copy(hs_hbm.at[srcv.at[j]], buf, sem)

    def gwait(buf):
        pltpu.make_async_copy(hs_hbm.at[srcv.at[0]], buf, sem).wait()

    def qbody(q, carry):
        base = off + q * QCH
        pltpu.sync_copy(src2_hbm.at[pl.ds(base, QCH)], srcv)
        pltpu.sync_copy(dst2_hbm.at[pl.ds(base, QCH)], dstv)

        gath(0, rows0)

        def body(jj, carry2):
            j0 = 2 * jj
            gwait(rows0)
            gath(j0 + 1, rows1)
            pltpu.sync_copy(rows0, y_sh.at[dstv.at[j0]], add=True)
            gwait(rows1)

            @pl.when(jj < QCH // 2 - 1)
            def _():
                gath(j0 + 2, rows0)

            pltpu.sync_copy(rows1, y_sh.at[dstv.at[j0 + 1]], add=True)
            return carry2

        lax.fori_loop(0, QCH // 2, body, 0)
        return carry

    lax.fori_loop(0, nq, qbody, 0)
    plsc.subcore_barrier()
    pltpu.sync_copy(y_sh.at[pl.ds(s * RPW, RPW)],
                    out_hbm.at[c].at[pl.ds(s * RPW, RPW)])


def _dis_block(deg_ref, i):
    p = deg_ref[...]
    deg = p[0, :, 0:1] + p[1, :, 0:1] + 1.0
    dis = lax.rsqrt(deg)
    rows = i * BM + lax.broadcasted_iota(jnp.int32, (BM, 1), 0)
    return jnp.where(rows < N, dis, 0.0)


def _k1_body(deg_ref, x_ref, w_ref, o_ref):
    i = pl.program_id(0)
    dis = _dis_block(deg_ref, i)
    o_ref[...] = dis * jnp.dot(x_ref[...], w_ref[...],
                               preferred_element_type=jnp.float32)


def _k2_body(deg_ref, y_ref, hs_ref, b_ref, o_ref, st_ref, acc_ref):
    i = pl.program_id(0)
    dis = _dis_block(deg_ref, i)
    o = dis * (y_ref[0] + y_ref[1] + hs_ref[...]) + b_ref[...]
    o_ref[...] = o

    @pl.when(i == 0)
    def _():
        acc_ref[...] = jnp.zeros_like(acc_ref)

    rows = i * BM + lax.broadcasted_iota(jnp.int32, (BM, 1), 0)
    om = jnp.where(rows < N, o, 0.0)
    acc_ref[0:1, :] += jnp.sum(om, axis=0, keepdims=True)
    acc_ref[1:2, :] += jnp.sum(om * om, axis=0, keepdims=True)

    @pl.when(i == NBLK - 1)
    def _():
        m = acc_ref[0:1, :] / N
        v = acc_ref[1:2, :] / N - m * m
        st_ref[...] = jnp.concatenate([m, lax.rsqrt(v + EPS)], axis=0)


def _bn_relu(o_ref, st_ref, g_ref, be_ref):
    st = st_ref[...]
    return jax.nn.relu((o_ref[...] - st[0:1, :]) * st[1:2, :]
                       * g_ref[...] + be_ref[...])


def _k3_body(deg_ref, o_ref, st_ref, g_ref, be_ref, w_ref, out_ref):
    i = pl.program_id(0)
    dis = _dis_block(deg_ref, i)
    h = _bn_relu(o_ref, st_ref, g_ref, be_ref)
    out_ref[...] = dis * jnp.dot(h, w_ref[...],
                                 preferred_element_type=jnp.float32)


def _kf_body(o_ref, st_ref, g_ref, be_ref, w1_ref, b1_ref, w2_ref, b2_ref,
             out_ref):
    h = _bn_relu(o_ref, st_ref, g_ref, be_ref)
    h = jax.nn.relu(jnp.dot(h, w1_ref[...],
                            preferred_element_type=jnp.float32) + b1_ref[...])
    o = jnp.dot(h, w2_ref[...],
                preferred_element_type=jnp.float32) + b2_ref[...]
    out_ref[...] = jax.nn.sigmoid(o)


def _full(shape):
    return pl.BlockSpec(shape, lambda i: tuple(0 for _ in shape))


_deg_spec = pl.BlockSpec((NC, BM, DEGW), lambda i: (0, i, 0))
_row_spec = pl.BlockSpec((BM, D), lambda i: (i, 0))


def _k1(degp, xpad, W):
    return pl.pallas_call(
        _k1_body,
        grid=(NBLK,),
        in_specs=[_deg_spec, _row_spec, _full((D, D))],
        out_specs=_row_spec,
        out_shape=jax.ShapeDtypeStruct((NPAD, D), jnp.float32),
    )(degp, xpad, W)


def _k2(degp, yp, hs, b):
    return pl.pallas_call(
        _k2_body,
        grid=(NBLK,),
        in_specs=[_deg_spec, pl.BlockSpec((NC, BM, D), lambda i: (0, i, 0)),
                  _row_spec, _full((1, D))],
        out_specs=[_row_spec, _full((2, D))],
        out_shape=[jax.ShapeDtypeStruct((NPAD, D), jnp.float32),
                   jax.ShapeDtypeStruct((2, D), jnp.float32)],
        scratch_shapes=[pltpu.VMEM((2, D), jnp.float32)],
    )(degp, yp, hs, b)


def _k3(degp, o, st, g, be, Wn):
    return pl.pallas_call(
        _k3_body,
        grid=(NBLK,),
        in_specs=[_deg_spec, _row_spec, _full((2, D)), _full((1, D)),
                  _full((1, D)), _full((D, D))],
        out_specs=_row_spec,
        out_shape=jax.ShapeDtypeStruct((NPAD, D), jnp.float32),
    )(degp, o, st, g, be, Wn)


def _kf(o, st, g, be, fc1_W, fc1_b, fc2_W, fc2_b):
    return pl.pallas_call(
        _kf_body,
        grid=(NBLK,),
        in_specs=[_row_spec, _full((2, D)), _full((1, D)), _full((1, D)),
                  _full((D, D // 2)), _full((1, D // 2)),
                  _full((D // 2, 1)), _full((1, 1))],
        out_specs=pl.BlockSpec((BM, 1), lambda i: (i, 0)),
        out_shape=jax.ShapeDtypeStruct((NPAD, 1), jnp.float32),
    )(o, st, g, be, fc1_W, fc1_b, fc2_W, fc2_b)


def kernel(x, edge_index, W1, b1, g1, be1, W2, b2, g2, be2, W3, b3, g3, be3,
           fc1_W, fc1_b, fc2_W, fc2_b):
    xpad = jnp.pad(x, ((0, NPAD - N), (0, 0)))
    pad = jnp.full((EPAD - E,), PADROW, jnp.int32)
    src2 = jnp.concatenate([edge_index[0], pad]).reshape(EROWS, CH)
    dst2 = jnp.concatenate([edge_index[1], pad]).reshape(EROWS, CH)
    zrowsD = jnp.zeros((RPW, D), jnp.float32)
    zrowsW = jnp.zeros((RPW, DEGW), jnp.float32)
    onesW = jnp.ones((CH, DEGW), jnp.float32)

    degp = _sc_degree(dst2, onesW, zrowsW)

    hs = _k1(degp, xpad, W1)
    params = [(b1, g1, be1), (b2, g2, be2), (b3, g3, be3)]
    nexts = [W2, W3, None]
    o = st = None
    for (b, g, be), Wn in zip(params, nexts):
        yp = _sc_spmm(hs, src2, dst2, zrowsD)
        o, st = _k2(degp, yp, hs, b.reshape(1, D))
        if Wn is not None:
            hs = _k3(degp, o, st, g.reshape(1, D), be.reshape(1, D), Wn)
    g3r, be3r = g3.reshape(1, D), be3.reshape(1, D)
    out = _kf(o, st, g3r, be3r, fc1_W, fc1_b.reshape(1, D // 2),
              fc2_W, fc2_b.reshape(1, 1))
    return out[:N]


# speedup vs baseline: 1.0442x; 1.0164x over previous
import functools

import jax
import jax.numpy as jnp
from jax import lax
from jax.experimental import pallas as pl
from jax.experimental.pallas import tpu as pltpu
from jax.experimental.pallas import tpu_sc as plsc

N = 10000
E = 320000
D = 128

NPAD = 10240
PADROW = 10200
NC = 2
NS = 16
NW = NC * NS
CH = 128
KCH = 80
EPW = CH * KCH
EPAD = EPW * NW
EROWS = EPAD // CH
FAST_CORE = 0
KFAST = 144
KSLOW = 16
QCH = 16
RPW = NPAD // NS

BM = 256
NBLK = NPAD // BM
EPS = 1e-5

_mesh = plsc.VectorSubcoreMesh(core_axis_name="c", subcore_axis_name="s")


def _make_deg(width):
    @functools.partial(
        pl.kernel,
        out_type=jax.ShapeDtypeStruct((NC, NPAD, width), jnp.float32),
        mesh=_mesh,
        scratch_types=[
            pltpu.VMEM((KCH, CH), jnp.int32),
            pltpu.VMEM((CH, width), jnp.float32),
            pltpu.VMEM_SHARED((NPAD, width), jnp.float32),
        ],
    )
    def _deg(dst2_hbm, ones_hbm, zeros_hbm, out_hbm, dstv, onesv, deg_sh):
        c = lax.axis_index("c")
        s = lax.axis_index("s")
        wid = s * NC + c
        pltpu.sync_copy(zeros_hbm, deg_sh.at[pl.ds(s * RPW, RPW)])
        pltpu.sync_copy(ones_hbm, onesv)
        pltpu.sync_copy(dst2_hbm.at[pl.ds(wid * KCH, KCH)], dstv)
        plsc.subcore_barrier()

        def body(j, carry):
            pltpu.sync_copy(onesv, deg_sh.at[dstv.at[j]], add=True)
            return carry

        lax.fori_loop(0, KCH, body, 0)
        plsc.subcore_barrier()
        pltpu.sync_copy(deg_sh.at[pl.ds(s * RPW, RPW)],
                        out_hbm.at[c].at[pl.ds(s * RPW, RPW)])

    return _deg


DEGW = 128
_sc_degree = _make_deg(DEGW)


@functools.partial(
    pl.kernel,
    out_type=jax.ShapeDtypeStruct((NC, NPAD, D), jnp.float32),
    mesh=_mesh,
    scratch_types=[
        pltpu.VMEM((QCH, CH), jnp.int32),
        pltpu.VMEM((QCH, CH), jnp.int32),
        pltpu.VMEM((CH, D), jnp.float32),
        pltpu.VMEM((CH, D), jnp.float32),
        pltpu.SemaphoreType.DMA,
        pltpu.VMEM_SHARED((NPAD, D), jnp.float32),
    ],
)
def _sc_spmm(hs_hbm, src2_hbm, dst2_hbm, zeros_hbm, out_hbm,
             srcv, dstv, rows0, rows1, sem, y_sh):
    c = lax.axis_index("c")
    s = lax.axis_index("s")
    pltpu.sync_copy(zeros_hbm, y_sh.at[pl.ds(s * RPW, RPW)])
    plsc.subcore_barrier()

    on_fast = c == FAST_CORE
    nq = jnp.where(on_fast, KFAST // QCH, KSLOW // QCH)
    off = jnp.where(on_fast, s * KFAST, NS * KFAST + s * KSLOW)

    def gath(j, buf):
        pltpu.async_copy(hs_hbm.at[srcv.at[j]], buf, sem)

    def gwait(buf):
        pltpu.make_async_copy(hs_hbm.at[srcv.at[0]], buf, sem).wait()

    def qbody(q, carry):
        base = off + q * QCH
        pltpu.sync_copy(src2_hbm.at[pl.ds(base, QCH)], srcv)
        pltpu.sync_copy(dst2_hbm.at[pl.ds(base, QCH)], dstv)

        gath(0, rows0)

        def body(jj, carry2):
            j0 = 2 * jj
            gwait(rows0)
            gath(j0 + 1, rows1)
            pltpu.sync_copy(rows0, y_sh.at[dstv.at[j0]], add=True)
            gwait(rows1)

            @pl.when(jj < QCH // 2 - 1)
            def _():
                gath(j0 + 2, rows0)

            pltpu.sync_copy(rows1, y_sh.at[dstv.at[j0 + 1]], add=True)
            return carry2

        lax.fori_loop(0, QCH // 2, body, 0)
        return carry

    lax.fori_loop(0, nq, qbody, 0)
    plsc.subcore_barrier()
    pltpu.sync_copy(y_sh.at[pl.ds(s * RPW, RPW)],
                    out_hbm.at[c].at[pl.ds(s * RPW, RPW)])


def _dis_block(deg_ref, i):
    p = deg_ref[...]
    deg = p[0, :, 0:1] + p[1, :, 0:1] + 1.0
    dis = lax.rsqrt(deg)
    rows = i * BM + lax.broadcasted_iota(jnp.int32, (BM, 1), 0)
    return jnp.where(rows < N, dis, 0.0)


def _k1_body(deg_ref, x_ref, w_ref, o_ref):
    i = pl.program_id(0)
    dis = _dis_block(deg_ref, i)
    o_ref[...] = dis * jnp.dot(x_ref[...], w_ref[...],
                               preferred_element_type=jnp.float32)


def _accum_stats(o, j, acc_ref):
    @pl.when(j == 0)
    def _():
        acc_ref[...] = jnp.zeros_like(acc_ref)

    rows = j * BM + lax.broadcasted_iota(jnp.int32, (BM, 1), 0)
    om = jnp.where(rows < N, o, 0.0)
    acc_ref[0:1, :] += jnp.sum(om, axis=0, keepdims=True)
    acc_ref[1:2, :] += jnp.sum(om * om, axis=0, keepdims=True)


def _bn_relu_scr(o_scr, j, acc_ref, g_ref, be_ref):
    m = acc_ref[0:1, :] / N
    rstd = lax.rsqrt(acc_ref[1:2, :] / N - m * m + EPS)
    o = o_scr[pl.ds(j * BM, BM), :]
    return jax.nn.relu((o - m) * rstd * g_ref[...] + be_ref[...])


def _phase0(deg_ref, y_ref, hs_ref, b_ref, j, o_scr, acc_ref):
    dis = _dis_block(deg_ref, j)
    o = dis * (y_ref[0] + y_ref[1] + hs_ref[...]) + b_ref[...]
    o_scr[pl.ds(j * BM, BM), :] = o
    _accum_stats(o, j, acc_ref)


def _k23_body(deg_ref, y_ref, hs_ref, b_ref, g_ref, be_ref, w_ref,
              out_ref, o_scr, acc_ref):
    i, j = pl.program_id(0), pl.program_id(1)

    @pl.when(i == 0)
    def _():
        _phase0(deg_ref, y_ref, hs_ref, b_ref, j, o_scr, acc_ref)

    @pl.when(i == 1)
    def _():
        h = _bn_relu_scr(o_scr, j, acc_ref, g_ref, be_ref)
        out_ref[...] = _dis_block(deg_ref, j) * jnp.dot(
            h, w_ref[...], preferred_element_type=jnp.float32)


def _k2f_body(deg_ref, y_ref, hs_ref, b_ref, g_ref, be_ref,
              w1_ref, b1_ref, w2_ref, b2_ref, out_ref, o_scr, acc_ref):
    i, j = pl.program_id(0), pl.program_id(1)

    @pl.when(i == 0)
    def _():
        _phase0(deg_ref, y_ref, hs_ref, b_ref, j, o_scr, acc_ref)

    @pl.when(i == 1)
    def _():
        h = _bn_relu_scr(o_scr, j, acc_ref, g_ref, be_ref)
        h = jax.nn.relu(jnp.dot(h, w1_ref[...],
                                preferred_element_type=jnp.float32)
                        + b1_ref[...])
        o = jnp.dot(h, w2_ref[...],
                    preferred_element_type=jnp.float32) + b2_ref[...]
        out_ref[...] = jax.nn.sigmoid(o)


def _full(shape):
    return pl.BlockSpec(shape, lambda *ids: tuple(0 for _ in shape))


_deg_spec = pl.BlockSpec((NC, BM, DEGW), lambda i: (0, i, 0))
_row_spec = pl.BlockSpec((BM, D), lambda i: (i, 0))

_deg2_spec = pl.BlockSpec((NC, BM, DEGW), lambda i, j: (0, j, 0))
_y2_spec = pl.BlockSpec((NC, BM, D),
                        lambda i, j: (0, jnp.where(i == 0, j, 0), 0))
_hs2_spec = pl.BlockSpec((BM, D), lambda i, j: (jnp.where(i == 0, j, 0), 0))
_scr23 = [pltpu.VMEM((NPAD, D), jnp.float32), pltpu.VMEM((2, D), jnp.float32)]


def _k1(degp, xpad, W):
    return pl.pallas_call(
        _k1_body,
        grid=(NBLK,),
        in_specs=[_deg_spec, _row_spec, _full((D, D))],
        out_specs=_row_spec,
        out_shape=jax.ShapeDtypeStruct((NPAD, D), jnp.float32),
    )(degp, xpad, W)


def _k23(degp, yp, hs, b, g, be, Wn):
    return pl.pallas_call(
        _k23_body,
        grid=(2, NBLK),
        in_specs=[_deg2_spec, _y2_spec, _hs2_spec, _full((1, D)),
                  _full((1, D)), _full((1, D)), _full((D, D))],
        out_specs=pl.BlockSpec((BM, D), lambda i, j: (j, 0)),
        out_shape=jax.ShapeDtypeStruct((NPAD, D), jnp.float32),
        scratch_shapes=_scr23,
    )(degp, yp, hs, b, g, be, Wn)


def _k2f(degp, yp, hs, b, g, be, fc1_W, fc1_b, fc2_W, fc2_b):
    return pl.pallas_call(
        _k2f_body,
        grid=(2, NBLK),
        in_specs=[_deg2_spec, _y2_spec, _hs2_spec, _full((1, D)),
                  _full((1, D)), _full((1, D)),
                  _full((D, D // 2)), _full((1, D // 2)),
                  _full((D // 2, 1)), _full((1, 1))],
        out_specs=pl.BlockSpec((BM, 1), lambda i, j: (j, 0)),
        out_shape=jax.ShapeDtypeStruct((NPAD, 1), jnp.float32),
        scratch_shapes=_scr23,
    )(degp, yp, hs, b, g, be, fc1_W, fc1_b, fc2_W, fc2_b)


def kernel(x, edge_index, W1, b1, g1, be1, W2, b2, g2, be2, W3, b3, g3, be3,
           fc1_W, fc1_b, fc2_W, fc2_b):
    xpad = jnp.pad(x, ((0, NPAD - N), (0, 0)))
    pad = jnp.full((EPAD - E,), PADROW, jnp.int32)
    src2 = jnp.concatenate([edge_index[0], pad]).reshape(EROWS, CH)
    dst2 = jnp.concatenate([edge_index[1], pad]).reshape(EROWS, CH)
    zrowsD = jnp.zeros((RPW, D), jnp.float32)
    zrowsW = jnp.zeros((RPW, DEGW), jnp.float32)
    onesW = jnp.ones((CH, DEGW), jnp.float32)

    degp = _sc_degree(dst2, onesW, zrowsW)

    hs = _k1(degp, xpad, W1)
    for b, g, be, Wn in [(b1, g1, be1, W2), (b2, g2, be2, W3)]:
        yp = _sc_spmm(hs, src2, dst2, zrowsD)
        hs = _k23(degp, yp, hs, b.reshape(1, D), g.reshape(1, D),
                  be.reshape(1, D), Wn)
    yp = _sc_spmm(hs, src2, dst2, zrowsD)
    out = _k2f(degp, yp, hs, b3.reshape(1, D), g3.reshape(1, D),
               be3.reshape(1, D), fc1_W, fc1_b.reshape(1, D // 2),
               fc2_W, fc2_b.reshape(1, 1))
    return out[:N]
